# baseline (speedup 1.0000x reference)
import functools
import jax
import jax.numpy as jnp
from jax import lax
from jax.experimental import pallas as pl
from jax.experimental.pallas import tpu as pltpu
from jax.experimental.pallas import tpu_sc as plsc

N, E = 10000, 320000
D_IN, D_EDGE, D_H, D_OUT = 128, 16, 32, 128

NC, NS = 2, 16
NW = NC * NS
C = 128
K = 80
EW = K * C
E_PAD = NW * EW
ROWS_PER_TILE = 640
NPAD = NS * ROWS_PER_TILE
DUMP = NPAD - 1


def _h_embed_body(x_ref, w_ref, b_ref, g_ref, beta_ref, h_ref):
    y = jnp.dot(x_ref[...], w_ref[...], preferred_element_type=jnp.float32)
    y = y + b_ref[...]
    m = jnp.mean(y, axis=0, keepdims=True)
    v = jnp.mean((y - m) * (y - m), axis=0, keepdims=True)
    hn = g_ref[...] * (y - m) / jnp.sqrt(v + 1e-5) + beta_ref[...]
    h_ref[...] = jnp.where(hn > 0, hn, 0.1 * hn)


def _egram_body(a_ref, g_ref, cs_ref):
    i = pl.program_id(0)
    a = a_ref[...]

    @pl.when(i == 0)
    def _():
        g_ref[...] = jnp.zeros_like(g_ref)
        cs_ref[...] = jnp.zeros_like(cs_ref)

    g_ref[...] += lax.dot_general(a, a, (((0,), (0,)), ((), ())),
                                  preferred_element_type=jnp.float32)
    cs_ref[...] += jnp.sum(a, axis=0, keepdims=True)


def _edge_table_body(a_ref, we_ref, be_ref, w_ref, b_ref, r_ref):
    y = jnp.dot(a_ref[...], we_ref[...], preferred_element_type=jnp.float32)
    y = y + be_ref[...]
    e = jnp.where(y > 0, y, 0.1 * y).astype(jnp.bfloat16)
    r_ref[...] = jnp.dot(e, w_ref[...], preferred_element_type=jnp.float32) + b_ref[...]


def _pq0_body(h_ref, wp_ref, wq_ref, p_ref, q_ref):
    h = h_ref[...]
    z = jnp.zeros((NPAD - N, 2 * D_H), jnp.float32)
    p_ref[...] = jnp.concatenate(
        [jnp.dot(h, wp_ref[...], preferred_element_type=jnp.float32), z], axis=0)
    q_ref[...] = jnp.concatenate(
        [jnp.dot(h, wq_ref[...], preferred_element_type=jnp.float32), z], axis=0)


def _pq1_body(h_ref, acc_ref, wp_ref, wq_ref, hn_ref, p_ref, q_ref):
    a = acc_ref[...]
    h = h_ref[...] + a[0, :N] + a[1, :N]
    hn_ref[...] = h
    z = jnp.zeros((NPAD - N, 2 * D_H), jnp.float32)
    p_ref[...] = jnp.concatenate(
        [jnp.dot(h, wp_ref[...], preferred_element_type=jnp.float32), z], axis=0)
    q_ref[...] = jnp.concatenate(
        [jnp.dot(h, wq_ref[...], preferred_element_type=jnp.float32), z], axis=0)


def _out_body(h_ref, acc_ref, w_ref, b_ref, g_ref, beta_ref, o_ref):
    a = acc_ref[...]
    h = h_ref[...] + a[0, :N] + a[1, :N]
    y = jnp.dot(h, w_ref[...], preferred_element_type=jnp.float32) + b_ref[...]
    m = jnp.mean(y, axis=0, keepdims=True)
    v = jnp.mean((y - m) * (y - m), axis=0, keepdims=True)
    o = g_ref[...] * (y - m) / jnp.sqrt(v + 1e-5) + beta_ref[...]
    o_ref[...] = jnp.where(o > 0, o, 0.1 * o)


def _msgpass_body(p_hbm, q_hbm, r_hbm, dst_hbm, src_hbm, out_hbm,
                  dst_all, src_all, pbuf, qbuf, rbuf, mbuf, zbuf, acc_sh,
                  sem_g0, sem_g1, sem_s0, sem_s1):
    cid = lax.axis_index("c")
    sid = lax.axis_index("s")
    wid = cid * NS + sid
    sem_g = (sem_g0, sem_g1)
    sem_s = (sem_s0, sem_s1)

    pltpu.sync_copy(dst_hbm.at[wid], dst_all)
    pltpu.sync_copy(src_hbm.at[wid], src_all)

    def _zrow(i, _):
        r = i // 2
        j = i - 2 * r
        zbuf[r, pl.ds(j * 16, 16)] = jnp.zeros((16,), jnp.float32)
        return 0
    lax.fori_loop(0, 2 * ROWS_PER_TILE, _zrow, 0)
    pltpu.sync_copy(zbuf, acc_sh.at[pl.ds(sid * ROWS_PER_TILE, ROWS_PER_TILE)])
    plsc.subcore_barrier()

    def _issue(k, b):
        ck = wid * K + k
        pltpu.async_copy(p_hbm.at[dst_all.at[k]], pbuf.at[b], sem_g[b])
        pltpu.async_copy(q_hbm.at[src_all.at[k]], qbuf.at[b], sem_g[b])
        pltpu.async_copy(r_hbm.at[pl.ds(ck * (C // 2), C // 2)], rbuf.at[b], sem_g[b])

    def _wait(k, b):
        pltpu.make_async_copy(p_hbm.at[dst_all.at[k]], pbuf.at[b], sem_g[b]).wait()
        pltpu.make_async_copy(q_hbm.at[src_all.at[k]], qbuf.at[b], sem_g[b]).wait()
        pltpu.make_async_copy(
            r_hbm.at[pl.ds((wid * K + k) * (C // 2), C // 2)], rbuf.at[b],
            sem_g[b]).wait()

    for b in range(2):
        _issue(b, b)

    def _pair(i, _):
        k0 = i * 2
        for b in range(2):
            k = k0 + b
            _wait(k, b)

            @pl.when(k >= 2)
            def _(k=k, b=b):
                pltpu.make_async_copy(
                    mbuf.at[b], acc_sh.at[dst_all.at[k]], sem_s[b]).wait()

            @plsc.parallel_loop(0, C, unroll=4)
            def _edge(c, b=b):
                rr = c // 2
                rc = (c - 2 * rr) * 64
                for j in range(2):
                    fo = pl.ds(j * 16, 16)
                    so = pl.ds(32 + j * 16, 16)
                    lf = pbuf[b, c, fo] + qbuf[b, c, fo] + rbuf[b, rr, pl.ds(rc + j * 16, 16)]
                    ls = pbuf[b, c, so] + qbuf[b, c, so] + rbuf[b, rr, pl.ds(rc + 32 + j * 16, 16)]
                    f = 1.0 / (1.0 + jnp.exp(-lf))
                    y = jnp.exp(-jnp.abs(ls))
                    t = y / (2.0 + y)
                    t2 = t * t
                    sp = jnp.maximum(ls, 0.0) + 2.0 * t * (
                        1.0 + t2 * (1.0 / 3.0 + t2 * (0.2 + t2 * (1.0 / 7.0))))
                    mbuf[b, c, fo] = f * sp

            pltpu.async_copy(
                mbuf.at[b], acc_sh.at[dst_all.at[k]], sem_s[b], add=True)

            @pl.when(k + 2 < K)
            def _(k=k, b=b):
                _issue(k + 2, b)
        return 0

    lax.fori_loop(0, K // 2, _pair, 0)
    for b in range(2):
        pltpu.make_async_copy(
            mbuf.at[b], acc_sh.at[dst_all.at[K - 2 + b]], sem_s[b]).wait()
    plsc.subcore_barrier()
    sl = pl.ds(sid * ROWS_PER_TILE, ROWS_PER_TILE)
    pltpu.sync_copy(acc_sh.at[sl], out_hbm.at[cid, sid])


_msgpass = functools.partial(
    pl.kernel,
    _msgpass_body,
    out_type=jax.ShapeDtypeStruct((NC, NS, ROWS_PER_TILE, D_H), jnp.float32),
    mesh=plsc.VectorSubcoreMesh(core_axis_name="c", subcore_axis_name="s"),
    scratch_types=[
        pltpu.VMEM((K, C), jnp.int32),
        pltpu.VMEM((K, C), jnp.int32),
        pltpu.VMEM((2, C, 2 * D_H), jnp.float32),
        pltpu.VMEM((2, C, 2 * D_H), jnp.float32),
        pltpu.VMEM((2, C // 2, 4 * D_H), jnp.float32),
        pltpu.VMEM((2, C, D_H), jnp.float32),
        pltpu.VMEM((ROWS_PER_TILE, D_H), jnp.float32),
        pltpu.VMEM_SHARED((NPAD, D_H), jnp.float32),
        pltpu.SemaphoreType.DMA,
        pltpu.SemaphoreType.DMA,
        pltpu.SemaphoreType.DMA,
        pltpu.SemaphoreType.DMA,
    ],
    compiler_params=pltpu.CompilerParams(use_tc_tiling_on_sc=False),
)()


@jax.jit
def kernel(x, edge_index, edge_attr, W_in, b_in, g_in, beta_in, W_e, b_e,
           g_e, beta_e, Wf0, bf0, Ws0, bs0, Wf1, bf1, Ws1, bs1, W_out,
           b_out, g_out, beta_out):
    f32 = jnp.float32
    b2 = lambda a: a.reshape(1, -1)

    h0 = pl.pallas_call(
        _h_embed_body,
        out_shape=jax.ShapeDtypeStruct((N, D_H), f32),
    )(x, W_in, b2(b_in), b2(g_in), b2(beta_in))

    PADW = EW - E // NW
    a_pad = jnp.concatenate(
        [edge_attr.reshape(NW, E // NW, D_EDGE),
         jnp.zeros((NW, PADW, D_EDGE), f32)], axis=1)
    dst3 = jnp.concatenate(
        [edge_index[1].reshape(NW, E // NW),
         jnp.full((NW, PADW), DUMP, jnp.int32)], axis=1).reshape(NW, K, C)
    src3 = jnp.concatenate(
        [edge_index[0].reshape(NW, E // NW),
         jnp.full((NW, PADW), DUMP, jnp.int32)], axis=1).reshape(NW, K, C)

    P8 = 8
    EC8 = 1024
    a8 = a_pad.reshape(E_PAD // P8, P8 * D_EDGE)
    G8, cs8 = pl.pallas_call(
        _egram_body,
        out_shape=[jax.ShapeDtypeStruct((P8 * D_EDGE, P8 * D_EDGE), f32),
                   jax.ShapeDtypeStruct((1, P8 * D_EDGE), f32)],
        grid=(E_PAD // P8 // EC8,),
        in_specs=[pl.BlockSpec((EC8, P8 * D_EDGE), lambda i: (i, 0))],
        out_specs=[pl.BlockSpec((P8 * D_EDGE, P8 * D_EDGE), lambda i: (0, 0)),
                   pl.BlockSpec((1, P8 * D_EDGE), lambda i: (0, 0))],
    )(a8)
    G = jnp.einsum('aiaj->ij', G8.reshape(P8, D_EDGE, P8, D_EDGE))
    cs = cs8.reshape(P8, D_EDGE).sum(axis=0)
    me = (cs @ W_e) / E + b_e
    Ey2 = (jnp.einsum('ij,ik,kj->j', W_e, G, W_e)
           + 2.0 * b_e * (cs @ W_e) + E * b_e * b_e) / E
    ve = Ey2 - me * me
    scale = g_e / jnp.sqrt(ve + 1e-5)
    We_t = W_e * scale
    be_t = (b_e - me) * scale + beta_e

    eye8 = jnp.eye(P8, dtype=f32)
    W8 = jnp.kron(eye8, We_t)
    b8 = jnp.tile(be_t, P8)
    w0 = jnp.kron(eye8, jnp.concatenate([Wf0[64:96], Ws0[64:96]], axis=1)).astype(jnp.bfloat16)
    b0 = jnp.tile(jnp.concatenate([bf0, bs0]), P8)
    w1 = jnp.kron(eye8, jnp.concatenate([Wf1[64:96], Ws1[64:96]], axis=1)).astype(jnp.bfloat16)
    b1 = jnp.tile(jnp.concatenate([bf1, bs1]), P8)

    def _table(w, b):
        r = pl.pallas_call(
            _edge_table_body,
            out_shape=jax.ShapeDtypeStruct((E_PAD // P8, P8 * 2 * D_H), f32),
            grid=(E_PAD // P8 // EC8,),
            in_specs=[
                pl.BlockSpec((EC8, P8 * D_EDGE), lambda i: (i, 0)),
                pl.BlockSpec((P8 * D_EDGE, P8 * D_H), lambda i: (0, 0)),
                pl.BlockSpec((1, P8 * D_H), lambda i: (0, 0)),
                pl.BlockSpec((P8 * D_H, P8 * 2 * D_H), lambda i: (0, 0)),
                pl.BlockSpec((1, P8 * 2 * D_H), lambda i: (0, 0)),
            ],
            out_specs=pl.BlockSpec((EC8, P8 * 2 * D_H), lambda i: (i, 0)),
        )(a8, W8, b2(b8), w, b2(b))
        return r.reshape(E_PAD // 2, 2 * 2 * D_H)

    R0 = _table(w0, b0)
    R1 = _table(w1, b1)

    wp0 = jnp.concatenate([Wf0[0:32], Ws0[0:32]], axis=1)
    wq0 = jnp.concatenate([Wf0[32:64], Ws0[32:64]], axis=1)
    P0, Q0 = pl.pallas_call(
        _pq0_body,
        out_shape=[jax.ShapeDtypeStruct((NPAD, 2 * D_H), f32)] * 2,
    )(h0, wp0, wq0)
    acc0 = _msgpass(P0, Q0, R0, dst3, src3).reshape(NC, NPAD, D_H)

    wp1 = jnp.concatenate([Wf1[0:32], Ws1[0:32]], axis=1)
    wq1 = jnp.concatenate([Wf1[32:64], Ws1[32:64]], axis=1)
    h1, P1, Q1 = pl.pallas_call(
        _pq1_body,
        out_shape=[jax.ShapeDtypeStruct((N, D_H), f32)] +
                  [jax.ShapeDtypeStruct((NPAD, 2 * D_H), f32)] * 2,
    )(h0, acc0, wp1, wq1)
    acc1 = _msgpass(P1, Q1, R1, dst3, src3).reshape(NC, NPAD, D_H)

    out = pl.pallas_call(
        _out_body,
        out_shape=jax.ShapeDtypeStruct((N, D_OUT), f32),
    )(h1, acc1, W_out, b2(b_out), b2(g_out), b2(beta_out))
    return out


# speedup vs baseline: 1.0149x; 1.0149x over previous
import functools
import jax
import jax.numpy as jnp
from jax import lax
from jax.experimental import pallas as pl
from jax.experimental.pallas import tpu as pltpu
from jax.experimental.pallas import tpu_sc as plsc

N, E = 10000, 320000
D_IN, D_EDGE, D_H, D_OUT = 128, 16, 32, 128

NC, NS = 2, 16
NW = NC * NS
C = 128
K = 80
EW = K * C
E_PAD = NW * EW
ROWS_PER_TILE = 640
NPAD = NS * ROWS_PER_TILE
DUMP = NPAD - 1


def _h_embed_body(x_ref, w_ref, b_ref, g_ref, beta_ref, h_ref):
    y = jnp.dot(x_ref[...], w_ref[...], preferred_element_type=jnp.float32)
    y = y + b_ref[...]
    m = jnp.mean(y, axis=0, keepdims=True)
    v = jnp.mean((y - m) * (y - m), axis=0, keepdims=True)
    hn = g_ref[...] * (y - m) / jnp.sqrt(v + 1e-5) + beta_ref[...]
    h_ref[...] = jnp.where(hn > 0, hn, 0.1 * hn)


def _egram_body(a_ref, g_ref, cs_ref):
    i = pl.program_id(0)
    a = a_ref[...]

    @pl.when(i == 0)
    def _():
        g_ref[...] = jnp.zeros_like(g_ref)
        cs_ref[...] = jnp.zeros_like(cs_ref)

    g_ref[...] += lax.dot_general(a, a, (((0,), (0,)), ((), ())),
                                  preferred_element_type=jnp.float32)
    cs_ref[...] += jnp.sum(a, axis=0, keepdims=True)


def _edge_table_body(a_ref, we_ref, be_ref, w_ref, b_ref, r_ref):
    y = jnp.dot(a_ref[...], we_ref[...], preferred_element_type=jnp.float32)
    y = y + be_ref[...]
    e = jnp.where(y > 0, y, 0.1 * y).astype(jnp.bfloat16)
    r_ref[...] = jnp.dot(e, w_ref[...], preferred_element_type=jnp.float32) + b_ref[...]


def _pq0_body(h_ref, wp_ref, wq_ref, p_ref, q_ref):
    h = h_ref[...]
    z = jnp.zeros((NPAD - N, 2 * D_H), jnp.float32)
    p_ref[...] = jnp.concatenate(
        [jnp.dot(h, wp_ref[...], preferred_element_type=jnp.float32), z], axis=0)
    q_ref[...] = jnp.concatenate(
        [jnp.dot(h, wq_ref[...], preferred_element_type=jnp.float32), z], axis=0)


def _pq1_body(h_ref, acc_ref, wp_ref, wq_ref, hn_ref, p_ref, q_ref):
    a = acc_ref[...]
    h = h_ref[...] + a[0, :N] + a[1, :N]
    hn_ref[...] = h
    z = jnp.zeros((NPAD - N, 2 * D_H), jnp.float32)
    p_ref[...] = jnp.concatenate(
        [jnp.dot(h, wp_ref[...], preferred_element_type=jnp.float32), z], axis=0)
    q_ref[...] = jnp.concatenate(
        [jnp.dot(h, wq_ref[...], preferred_element_type=jnp.float32), z], axis=0)


def _out_body(h_ref, acc_ref, w_ref, b_ref, g_ref, beta_ref, o_ref):
    a = acc_ref[...]
    h = h_ref[...] + a[0, :N] + a[1, :N]
    y = jnp.dot(h, w_ref[...], preferred_element_type=jnp.float32) + b_ref[...]
    m = jnp.mean(y, axis=0, keepdims=True)
    v = jnp.mean((y - m) * (y - m), axis=0, keepdims=True)
    o = g_ref[...] * (y - m) / jnp.sqrt(v + 1e-5) + beta_ref[...]
    o_ref[...] = jnp.where(o > 0, o, 0.1 * o)


def _msgpass_body(p_hbm, q_hbm, r_hbm, dst_hbm, src_hbm, out_hbm,
                  dst_all, src_all, pbuf, qbuf, rbuf, mbuf, zbuf, acc_sh,
                  sem_g0, sem_g1, sem_s0, sem_s1):
    cid = lax.axis_index("c")
    sid = lax.axis_index("s")
    wid = cid * NS + sid
    sem_g = (sem_g0, sem_g1)
    sem_s = (sem_s0, sem_s1)

    pltpu.sync_copy(dst_hbm.at[wid], dst_all)
    pltpu.sync_copy(src_hbm.at[wid], src_all)

    def _zrow(i, _):
        r = i // 2
        j = i - 2 * r
        zbuf[r, pl.ds(j * 16, 16)] = jnp.zeros((16,), jnp.float32)
        return 0
    lax.fori_loop(0, 2 * ROWS_PER_TILE, _zrow, 0)
    pltpu.sync_copy(zbuf, acc_sh.at[pl.ds(sid * ROWS_PER_TILE, ROWS_PER_TILE)])
    plsc.subcore_barrier()

    RW = C * 2 * D_H

    def _issue(k, b):
        ck = wid * K + k
        pltpu.async_copy(p_hbm.at[dst_all.at[k]], pbuf.at[b], sem_g[b])
        pltpu.async_copy(q_hbm.at[src_all.at[k]], qbuf.at[b], sem_g[b])
        pltpu.async_copy(r_hbm.at[pl.ds(ck * RW, RW)], rbuf.at[b], sem_g[b])

    def _wait(k, b):
        pltpu.make_async_copy(p_hbm.at[dst_all.at[k]], pbuf.at[b], sem_g[b]).wait()
        pltpu.make_async_copy(q_hbm.at[src_all.at[k]], qbuf.at[b], sem_g[b]).wait()
        pltpu.make_async_copy(
            r_hbm.at[pl.ds((wid * K + k) * RW, RW)], rbuf.at[b],
            sem_g[b]).wait()

    for b in range(2):
        _issue(b, b)

    def _pair(i, _):
        k0 = i * 2
        for b in range(2):
            k = k0 + b
            _wait(k, b)

            @pl.when(k >= 2)
            def _(k=k, b=b):
                pltpu.make_async_copy(
                    mbuf.at[b], acc_sh.at[dst_all.at[k]], sem_s[b]).wait()

            @plsc.parallel_loop(0, C, unroll=8)
            def _edge(c, b=b):
                rbase = c * 2 * D_H
                for j in range(2):
                    fo = pl.ds(j * 16, 16)
                    so = pl.ds(32 + j * 16, 16)
                    lf = pbuf[b, c, fo] + qbuf[b, c, fo] + rbuf[b, pl.ds(rbase + j * 16, 16)]
                    ls = pbuf[b, c, so] + qbuf[b, c, so] + rbuf[b, pl.ds(rbase + 32 + j * 16, 16)]
                    f = 1.0 / (1.0 + jnp.exp(-lf))
                    y = jnp.exp(-jnp.abs(ls))
                    t = y / (2.0 + y)
                    t2 = t * t
                    sp = jnp.maximum(ls, 0.0) + 2.0 * t * (
                        1.0 + t2 * (1.0 / 3.0 + t2 * (0.2 + t2 * (1.0 / 7.0))))
                    mbuf[b, c, fo] = f * sp

            pltpu.async_copy(
                mbuf.at[b], acc_sh.at[dst_all.at[k]], sem_s[b], add=True)

            @pl.when(k + 2 < K)
            def _(k=k, b=b):
                _issue(k + 2, b)
        return 0

    lax.fori_loop(0, K // 2, _pair, 0)
    for b in range(2):
        pltpu.make_async_copy(
            mbuf.at[b], acc_sh.at[dst_all.at[K - 2 + b]], sem_s[b]).wait()
    plsc.subcore_barrier()
    sl = pl.ds(sid * ROWS_PER_TILE, ROWS_PER_TILE)
    pltpu.sync_copy(acc_sh.at[sl], out_hbm.at[cid, sid])


_msgpass = functools.partial(
    pl.kernel,
    _msgpass_body,
    out_type=jax.ShapeDtypeStruct((NC, NS, ROWS_PER_TILE, D_H), jnp.float32),
    mesh=plsc.VectorSubcoreMesh(core_axis_name="c", subcore_axis_name="s"),
    scratch_types=[
        pltpu.VMEM((K, C), jnp.int32),
        pltpu.VMEM((K, C), jnp.int32),
        pltpu.VMEM((2, C, 2 * D_H), jnp.float32),
        pltpu.VMEM((2, C, 2 * D_H), jnp.float32),
        pltpu.VMEM((2, C * 2 * D_H), jnp.float32),
        pltpu.VMEM((2, C, D_H), jnp.float32),
        pltpu.VMEM((ROWS_PER_TILE, D_H), jnp.float32),
        pltpu.VMEM_SHARED((NPAD, D_H), jnp.float32),
        pltpu.SemaphoreType.DMA,
        pltpu.SemaphoreType.DMA,
        pltpu.SemaphoreType.DMA,
        pltpu.SemaphoreType.DMA,
    ],
    compiler_params=pltpu.CompilerParams(use_tc_tiling_on_sc=False),
)()


@jax.jit
def kernel(x, edge_index, edge_attr, W_in, b_in, g_in, beta_in, W_e, b_e,
           g_e, beta_e, Wf0, bf0, Ws0, bs0, Wf1, bf1, Ws1, bs1, W_out,
           b_out, g_out, beta_out):
    f32 = jnp.float32
    b2 = lambda a: a.reshape(1, -1)

    h0 = pl.pallas_call(
        _h_embed_body,
        out_shape=jax.ShapeDtypeStruct((N, D_H), f32),
    )(x, W_in, b2(b_in), b2(g_in), b2(beta_in))

    PADW = EW - E // NW
    a_pad = jnp.concatenate(
        [edge_attr.reshape(NW, E // NW, D_EDGE),
         jnp.zeros((NW, PADW, D_EDGE), f32)], axis=1)
    dst3 = jnp.concatenate(
        [edge_index[1].reshape(NW, E // NW),
         jnp.full((NW, PADW), DUMP, jnp.int32)], axis=1).reshape(NW, K, C)
    src3 = jnp.concatenate(
        [edge_index[0].reshape(NW, E // NW),
         jnp.full((NW, PADW), DUMP, jnp.int32)], axis=1).reshape(NW, K, C)

    P8 = 8
    EC8 = 1024
    a8 = a_pad.reshape(E_PAD // P8, P8 * D_EDGE)
    G8, cs8 = pl.pallas_call(
        _egram_body,
        out_shape=[jax.ShapeDtypeStruct((P8 * D_EDGE, P8 * D_EDGE), f32),
                   jax.ShapeDtypeStruct((1, P8 * D_EDGE), f32)],
        grid=(E_PAD // P8 // EC8,),
        in_specs=[pl.BlockSpec((EC8, P8 * D_EDGE), lambda i: (i, 0))],
        out_specs=[pl.BlockSpec((P8 * D_EDGE, P8 * D_EDGE), lambda i: (0, 0)),
                   pl.BlockSpec((1, P8 * D_EDGE), lambda i: (0, 0))],
    )(a8)
    G = jnp.einsum('aiaj->ij', G8.reshape(P8, D_EDGE, P8, D_EDGE))
    cs = cs8.reshape(P8, D_EDGE).sum(axis=0)
    me = (cs @ W_e) / E + b_e
    Ey2 = (jnp.einsum('ij,ik,kj->j', W_e, G, W_e)
           + 2.0 * b_e * (cs @ W_e) + E * b_e * b_e) / E
    ve = Ey2 - me * me
    scale = g_e / jnp.sqrt(ve + 1e-5)
    We_t = W_e * scale
    be_t = (b_e - me) * scale + beta_e

    eye8 = jnp.eye(P8, dtype=f32)
    W8 = jnp.kron(eye8, We_t)
    b8 = jnp.tile(be_t, P8)
    w0 = jnp.kron(eye8, jnp.concatenate([Wf0[64:96], Ws0[64:96]], axis=1)).astype(jnp.bfloat16)
    b0 = jnp.tile(jnp.concatenate([bf0, bs0]), P8)
    w1 = jnp.kron(eye8, jnp.concatenate([Wf1[64:96], Ws1[64:96]], axis=1)).astype(jnp.bfloat16)
    b1 = jnp.tile(jnp.concatenate([bf1, bs1]), P8)

    def _table(w, b):
        r = pl.pallas_call(
            _edge_table_body,
            out_shape=jax.ShapeDtypeStruct((E_PAD // P8, P8 * 2 * D_H), f32),
            grid=(E_PAD // P8 // EC8,),
            in_specs=[
                pl.BlockSpec((EC8, P8 * D_EDGE), lambda i: (i, 0)),
                pl.BlockSpec((P8 * D_EDGE, P8 * D_H), lambda i: (0, 0)),
                pl.BlockSpec((1, P8 * D_H), lambda i: (0, 0)),
                pl.BlockSpec((P8 * D_H, P8 * 2 * D_H), lambda i: (0, 0)),
                pl.BlockSpec((1, P8 * 2 * D_H), lambda i: (0, 0)),
            ],
            out_specs=pl.BlockSpec((EC8, P8 * 2 * D_H), lambda i: (i, 0)),
        )(a8, W8, b2(b8), w, b2(b))
        return r.reshape(E_PAD * 2 * D_H)

    R0 = _table(w0, b0)
    R1 = _table(w1, b1)

    wp0 = jnp.concatenate([Wf0[0:32], Ws0[0:32]], axis=1)
    wq0 = jnp.concatenate([Wf0[32:64], Ws0[32:64]], axis=1)
    P0, Q0 = pl.pallas_call(
        _pq0_body,
        out_shape=[jax.ShapeDtypeStruct((NPAD, 2 * D_H), f32)] * 2,
    )(h0, wp0, wq0)
    acc0 = _msgpass(P0, Q0, R0, dst3, src3).reshape(NC, NPAD, D_H)

    wp1 = jnp.concatenate([Wf1[0:32], Ws1[0:32]], axis=1)
    wq1 = jnp.concatenate([Wf1[32:64], Ws1[32:64]], axis=1)
    h1, P1, Q1 = pl.pallas_call(
        _pq1_body,
        out_shape=[jax.ShapeDtypeStruct((N, D_H), f32)] +
                  [jax.ShapeDtypeStruct((NPAD, 2 * D_H), f32)] * 2,
    )(h0, acc0, wp1, wq1)
    acc1 = _msgpass(P1, Q1, R1, dst3, src3).reshape(NC, NPAD, D_H)

    out = pl.pallas_call(
        _out_body,
        out_shape=jax.ShapeDtypeStruct((N, D_OUT), f32),
    )(h1, acc1, W_out, b2(b_out), b2(g_out), b2(beta_out))
    return out


# speedup vs baseline: 1.6311x; 1.6071x over previous
import functools
import jax
import jax.numpy as jnp
from jax import lax
from jax.experimental import pallas as pl
from jax.experimental.pallas import tpu as pltpu
from jax.experimental.pallas import tpu_sc as plsc

N, E = 10000, 320000
D_IN, D_EDGE, D_H, D_OUT = 128, 16, 32, 128

NC, NS = 2, 16
NW = NC * NS
C = 125
K = 80
ROWS_PER_TILE = 640
NPAD = NS * ROWS_PER_TILE


def _h_embed_body(x_ref, w_ref, b_ref, g_ref, beta_ref, h_ref):
    y = jnp.dot(x_ref[...], w_ref[...], preferred_element_type=jnp.float32)
    y = y + b_ref[...]
    m = jnp.mean(y, axis=0, keepdims=True)
    v = jnp.mean((y - m) * (y - m), axis=0, keepdims=True)
    hn = g_ref[...] * (y - m) / jnp.sqrt(v + 1e-5) + beta_ref[...]
    h_ref[...] = jnp.where(hn > 0, hn, 0.1 * hn)


def _egram_body(a_ref, g_ref, cs_ref):
    i = pl.program_id(0)
    a = a_ref[...]

    @pl.when(i == 0)
    def _():
        g_ref[...] = jnp.zeros_like(g_ref)
        cs_ref[...] = jnp.zeros_like(cs_ref)

    g_ref[...] += lax.dot_general(a, a, (((0,), (0,)), ((), ())),
                                  preferred_element_type=jnp.float32)
    cs_ref[...] += jnp.sum(a, axis=0, keepdims=True)


def _edge_table_body(a_ref, we_ref, be_ref, w_ref, b_ref, r_ref):
    y = jnp.dot(a_ref[...], we_ref[...], preferred_element_type=jnp.float32)
    y = y + be_ref[...]
    e = jnp.where(y > 0, y, 0.1 * y).astype(jnp.bfloat16)
    r_ref[...] = jnp.dot(e, w_ref[...], preferred_element_type=jnp.float32) + b_ref[...]


def _pq0_body(h_ref, wp_ref, wq_ref, p_ref, q_ref):
    h = h_ref[...]
    p_ref[...] = jnp.dot(h, wp_ref[...], preferred_element_type=jnp.float32)
    q_ref[...] = jnp.dot(h, wq_ref[...], preferred_element_type=jnp.float32)


def _pq1_body(h_ref, acc_ref, wp_ref, wq_ref, hn_ref, p_ref, q_ref):
    a = acc_ref[...]
    h = h_ref[...] + a[0, :N] + a[1, :N]
    hn_ref[...] = h
    p_ref[...] = jnp.dot(h, wp_ref[...], preferred_element_type=jnp.float32)
    q_ref[...] = jnp.dot(h, wq_ref[...], preferred_element_type=jnp.float32)


def _out_body(h_ref, acc_ref, w_ref, b_ref, g_ref, beta_ref, o_ref):
    a = acc_ref[...]
    h = h_ref[...] + a[0, :N] + a[1, :N]
    y = jnp.dot(h, w_ref[...], preferred_element_type=jnp.float32) + b_ref[...]
    m = jnp.mean(y, axis=0, keepdims=True)
    v = jnp.mean((y - m) * (y - m), axis=0, keepdims=True)
    o = g_ref[...] * (y - m) / jnp.sqrt(v + 1e-5) + beta_ref[...]
    o_ref[...] = jnp.where(o > 0, o, 0.1 * o)


def _msgpass_body(p_hbm, q_hbm, r_hbm, dst_hbm, src_hbm, out_hbm,
                  dst_all, src_all, pbuf, qbuf, rbuf, mbuf, zbuf, acc_sh,
                  sem_g0, sem_g1, sem_s0, sem_s1):
    cid = lax.axis_index("c")
    sid = lax.axis_index("s")
    wid = cid * NS + sid
    sem_g = (sem_g0, sem_g1)
    sem_s = (sem_s0, sem_s1)

    pltpu.sync_copy(dst_hbm.at[pl.ds(wid * K, K)], dst_all)
    pltpu.sync_copy(src_hbm.at[pl.ds(wid * K, K)], src_all)

    def _zrow(i, _):
        r = i // 2
        j = i - 2 * r
        zbuf[r, pl.ds(j * 16, 16)] = jnp.zeros((16,), jnp.float32)
        return 0
    lax.fori_loop(0, 2 * ROWS_PER_TILE, _zrow, 0)
    pltpu.sync_copy(zbuf, acc_sh.at[pl.ds(sid * ROWS_PER_TILE, ROWS_PER_TILE)])
    plsc.subcore_barrier()

    def _issue(k, b):
        ck = wid * K + k
        pltpu.async_copy(p_hbm.at[dst_all.at[k, 0]], pbuf.at[b], sem_g[b])
        pltpu.async_copy(q_hbm.at[src_all.at[k, 0]], qbuf.at[b], sem_g[b])
        pltpu.async_copy(r_hbm.at[ck], rbuf.at[b], sem_g[b])

    def _wait(k, b):
        pltpu.make_async_copy(p_hbm.at[dst_all.at[k, 0]], pbuf.at[b], sem_g[b]).wait()
        pltpu.make_async_copy(q_hbm.at[src_all.at[k, 0]], qbuf.at[b], sem_g[b]).wait()
        pltpu.make_async_copy(r_hbm.at[wid * K + k], rbuf.at[b], sem_g[b]).wait()

    for b in range(2):
        _issue(b, b)

    def _pair(i, _):
        k0 = i * 2
        for b in range(2):
            k = k0 + b
            _wait(k, b)

            @pl.when(k >= 2)
            def _(k=k, b=b):
                pltpu.make_async_copy(
                    mbuf.at[b], acc_sh.at[dst_all.at[k, 0]], sem_s[b]).wait()

            @plsc.parallel_loop(0, C, unroll=5)
            def _edge(c, b=b):
                for j in range(2):
                    fo = pl.ds(j * 16, 16)
                    so = pl.ds(32 + j * 16, 16)
                    lf = pbuf[b, c, fo] + qbuf[b, c, fo] + rbuf[b, c, fo]
                    ls = pbuf[b, c, so] + qbuf[b, c, so] + rbuf[b, c, so]
                    f = 1.0 / (1.0 + jnp.exp(-lf))
                    y = jnp.exp(-jnp.abs(ls))
                    t = y / (2.0 + y)
                    t2 = t * t
                    sp = jnp.maximum(ls, 0.0) + 2.0 * t * (
                        1.0 + t2 * (1.0 / 3.0 + t2 * (0.2 + t2 * (1.0 / 7.0))))
                    mbuf[b, c, fo] = f * sp

            pltpu.async_copy(
                mbuf.at[b], acc_sh.at[dst_all.at[k, 0]], sem_s[b], add=True)

            @pl.when(k + 2 < K)
            def _(k=k, b=b):
                _issue(k + 2, b)
        return 0

    lax.fori_loop(0, K // 2, _pair, 0)
    for b in range(2):
        pltpu.make_async_copy(
            mbuf.at[b], acc_sh.at[dst_all.at[K - 2 + b, 0]], sem_s[b]).wait()
    plsc.subcore_barrier()
    sl = pl.ds(sid * ROWS_PER_TILE, ROWS_PER_TILE)
    pltpu.sync_copy(acc_sh.at[sl], out_hbm.at[cid, sid])


_msgpass = functools.partial(
    pl.kernel,
    _msgpass_body,
    out_type=jax.ShapeDtypeStruct((NC, NS, ROWS_PER_TILE, D_H), jnp.float32),
    mesh=plsc.VectorSubcoreMesh(core_axis_name="c", subcore_axis_name="s"),
    scratch_types=[
        pltpu.VMEM((K, 1, C), jnp.int32),
        pltpu.VMEM((K, 1, C), jnp.int32),
        pltpu.VMEM((2, C, 2 * D_H), jnp.float32),
        pltpu.VMEM((2, C, 2 * D_H), jnp.float32),
        pltpu.VMEM((2, C, 2 * D_H), jnp.float32),
        pltpu.VMEM((2, C, D_H), jnp.float32),
        pltpu.VMEM((ROWS_PER_TILE, D_H), jnp.float32),
        pltpu.VMEM_SHARED((NPAD, D_H), jnp.float32),
        pltpu.SemaphoreType.DMA,
        pltpu.SemaphoreType.DMA,
        pltpu.SemaphoreType.DMA,
        pltpu.SemaphoreType.DMA,
    ],
    compiler_params=pltpu.CompilerParams(use_tc_tiling_on_sc=False),
)()


@jax.jit
def kernel(x, edge_index, edge_attr, W_in, b_in, g_in, beta_in, W_e, b_e,
           g_e, beta_e, Wf0, bf0, Ws0, bs0, Wf1, bf1, Ws1, bs1, W_out,
           b_out, g_out, beta_out):
    f32 = jnp.float32
    b2 = lambda a: a.reshape(1, -1)

    h0 = pl.pallas_call(
        _h_embed_body,
        out_shape=jax.ShapeDtypeStruct((N, D_H), f32),
    )(x, W_in, b2(b_in), b2(g_in), b2(beta_in))

    dst3 = edge_index[1].reshape(NW * K, 1, C)
    src3 = edge_index[0].reshape(NW * K, 1, C)

    P8 = 8
    EC8 = 800
    a8 = edge_attr.reshape(E // P8, P8 * D_EDGE)
    G8, cs8 = pl.pallas_call(
        _egram_body,
        out_shape=[jax.ShapeDtypeStruct((P8 * D_EDGE, P8 * D_EDGE), f32),
                   jax.ShapeDtypeStruct((1, P8 * D_EDGE), f32)],
        grid=(E // P8 // EC8,),
        in_specs=[pl.BlockSpec((EC8, P8 * D_EDGE), lambda i: (i, 0))],
        out_specs=[pl.BlockSpec((P8 * D_EDGE, P8 * D_EDGE), lambda i: (0, 0)),
                   pl.BlockSpec((1, P8 * D_EDGE), lambda i: (0, 0))],
    )(a8)
    G = jnp.einsum('aiaj->ij', G8.reshape(P8, D_EDGE, P8, D_EDGE))
    cs = cs8.reshape(P8, D_EDGE).sum(axis=0)
    me = (cs @ W_e) / E + b_e
    Ey2 = (jnp.einsum('ij,ik,kj->j', W_e, G, W_e)
           + 2.0 * b_e * (cs @ W_e) + E * b_e * b_e) / E
    ve = Ey2 - me * me
    scale = g_e / jnp.sqrt(ve + 1e-5)
    We_t = W_e * scale
    be_t = (b_e - me) * scale + beta_e

    eye8 = jnp.eye(P8, dtype=f32)
    W8 = jnp.kron(eye8, We_t)
    b8 = jnp.tile(be_t, P8)
    w0 = jnp.kron(eye8, jnp.concatenate([Wf0[64:96], Ws0[64:96]], axis=1)).astype(jnp.bfloat16)
    b0 = jnp.tile(jnp.concatenate([bf0, bs0]), P8)
    w1 = jnp.kron(eye8, jnp.concatenate([Wf1[64:96], Ws1[64:96]], axis=1)).astype(jnp.bfloat16)
    b1 = jnp.tile(jnp.concatenate([bf1, bs1]), P8)

    def _table(w, b):
        r = pl.pallas_call(
            _edge_table_body,
            out_shape=jax.ShapeDtypeStruct((E // P8, P8 * 2 * D_H), f32),
            grid=(E // P8 // EC8,),
            in_specs=[
                pl.BlockSpec((EC8, P8 * D_EDGE), lambda i: (i, 0)),
                pl.BlockSpec((P8 * D_EDGE, P8 * D_H), lambda i: (0, 0)),
                pl.BlockSpec((1, P8 * D_H), lambda i: (0, 0)),
                pl.BlockSpec((P8 * D_H, P8 * 2 * D_H), lambda i: (0, 0)),
                pl.BlockSpec((1, P8 * 2 * D_H), lambda i: (0, 0)),
            ],
            out_specs=pl.BlockSpec((EC8, P8 * 2 * D_H), lambda i: (i, 0)),
        )(a8, W8, b2(b8), w, b2(b))
        return r.reshape(NW * K, C, 2 * D_H)

    R0 = _table(w0, b0)
    R1 = _table(w1, b1)

    wp0 = jnp.concatenate([Wf0[0:32], Ws0[0:32]], axis=1)
    wq0 = jnp.concatenate([Wf0[32:64], Ws0[32:64]], axis=1)
    P0, Q0 = pl.pallas_call(
        _pq0_body,
        out_shape=[jax.ShapeDtypeStruct((N, 2 * D_H), f32)] * 2,
    )(h0, wp0, wq0)
    acc0 = _msgpass(P0, Q0, R0, dst3, src3).reshape(NC, NPAD, D_H)

    wp1 = jnp.concatenate([Wf1[0:32], Ws1[0:32]], axis=1)
    wq1 = jnp.concatenate([Wf1[32:64], Ws1[32:64]], axis=1)
    h1, P1, Q1 = pl.pallas_call(
        _pq1_body,
        out_shape=[jax.ShapeDtypeStruct((N, D_H), f32)] +
                  [jax.ShapeDtypeStruct((N, 2 * D_H), f32)] * 2,
    )(h0, acc0, wp1, wq1)
    acc1 = _msgpass(P1, Q1, R1, dst3, src3).reshape(NC, NPAD, D_H)

    out = pl.pallas_call(
        _out_body,
        out_shape=jax.ShapeDtypeStruct((N, D_OUT), f32),
    )(h1, acc1, W_out, b2(b_out), b2(g_out), b2(beta_out))
    return out


# speedup vs baseline: 1.6316x; 1.0003x over previous
import functools
import jax
import jax.numpy as jnp
from jax import lax
from jax.experimental import pallas as pl
from jax.experimental.pallas import tpu as pltpu
from jax.experimental.pallas import tpu_sc as plsc

N, E = 10000, 320000
D_IN, D_EDGE, D_H, D_OUT = 128, 16, 32, 128

NC, NS = 2, 16
NW = NC * NS
C = 125
K = 80
ROWS_PER_TILE = 640
NPAD = NS * ROWS_PER_TILE


def _h_embed_body(x_ref, w_ref, b_ref, g_ref, beta_ref, h_ref):
    y = jnp.dot(x_ref[...], w_ref[...], preferred_element_type=jnp.float32)
    y = y + b_ref[...]
    m = jnp.mean(y, axis=0, keepdims=True)
    v = jnp.mean((y - m) * (y - m), axis=0, keepdims=True)
    hn = g_ref[...] * (y - m) / jnp.sqrt(v + 1e-5) + beta_ref[...]
    h_ref[...] = jnp.where(hn > 0, hn, 0.1 * hn)


def _egram_body(a_ref, g_ref, cs_ref):
    i = pl.program_id(0)
    a = a_ref[...]

    @pl.when(i == 0)
    def _():
        g_ref[...] = jnp.zeros_like(g_ref)
        cs_ref[...] = jnp.zeros_like(cs_ref)

    g_ref[...] += lax.dot_general(a, a, (((0,), (0,)), ((), ())),
                                  preferred_element_type=jnp.float32)
    cs_ref[...] += jnp.sum(a, axis=0, keepdims=True)


def _edge_table_body(a_ref, we_ref, be_ref, w_ref, b_ref, r_ref):
    y = jnp.dot(a_ref[...], we_ref[...], preferred_element_type=jnp.float32)
    y = y + be_ref[...]
    e = jnp.where(y > 0, y, 0.1 * y).astype(jnp.bfloat16)
    r_ref[...] = jnp.dot(e, w_ref[...], preferred_element_type=jnp.float32) + b_ref[...]


def _pq0_body(h_ref, wp_ref, wq_ref, p_ref, q_ref):
    h = h_ref[...]
    p_ref[...] = jnp.dot(h, wp_ref[...], preferred_element_type=jnp.float32)
    q_ref[...] = jnp.dot(h, wq_ref[...], preferred_element_type=jnp.float32)


def _pq1_body(h_ref, acc_ref, wp_ref, wq_ref, hn_ref, p_ref, q_ref):
    a = acc_ref[...]
    h = h_ref[...] + a[0, :N] + a[1, :N]
    hn_ref[...] = h
    p_ref[...] = jnp.dot(h, wp_ref[...], preferred_element_type=jnp.float32)
    q_ref[...] = jnp.dot(h, wq_ref[...], preferred_element_type=jnp.float32)


def _out_body(h_ref, acc_ref, w_ref, b_ref, g_ref, beta_ref, o_ref):
    a = acc_ref[...]
    h = h_ref[...] + a[0, :N] + a[1, :N]
    y = jnp.dot(h, w_ref[...], preferred_element_type=jnp.float32) + b_ref[...]
    m = jnp.mean(y, axis=0, keepdims=True)
    v = jnp.mean((y - m) * (y - m), axis=0, keepdims=True)
    o = g_ref[...] * (y - m) / jnp.sqrt(v + 1e-5) + beta_ref[...]
    o_ref[...] = jnp.where(o > 0, o, 0.1 * o)


def _msgpass_body(p_hbm, q_hbm, r_hbm, dst_hbm, src_hbm, out_hbm,
                  dst_all, src_all, pbuf, qbuf, rbuf, mbuf, zbuf, acc_sh,
                  sem_g0, sem_g1, sem_s0, sem_s1):
    cid = lax.axis_index("c")
    sid = lax.axis_index("s")
    wid = cid * NS + sid
    sem_g = (sem_g0, sem_g1)
    sem_s = (sem_s0, sem_s1)

    pltpu.sync_copy(dst_hbm.at[pl.ds(wid * K, K)], dst_all)
    pltpu.sync_copy(src_hbm.at[pl.ds(wid * K, K)], src_all)

    def _zrow(i, _):
        r = i // 2
        j = i - 2 * r
        zbuf[r, pl.ds(j * 16, 16)] = jnp.zeros((16,), jnp.float32)
        return 0
    lax.fori_loop(0, 2 * ROWS_PER_TILE, _zrow, 0)
    pltpu.sync_copy(zbuf, acc_sh.at[pl.ds(sid * ROWS_PER_TILE, ROWS_PER_TILE)])
    plsc.subcore_barrier()

    def _issue(k, b):
        ck = wid * K + k
        pltpu.async_copy(p_hbm.at[dst_all.at[k, 0]], pbuf.at[b], sem_g[b])
        pltpu.async_copy(q_hbm.at[src_all.at[k, 0]], qbuf.at[b], sem_g[b])
        pltpu.async_copy(r_hbm.at[ck], rbuf.at[b], sem_g[b])

    def _wait(k, b):
        pltpu.make_async_copy(p_hbm.at[dst_all.at[k, 0]], pbuf.at[b], sem_g[b]).wait()
        pltpu.make_async_copy(q_hbm.at[src_all.at[k, 0]], qbuf.at[b], sem_g[b]).wait()
        pltpu.make_async_copy(r_hbm.at[wid * K + k], rbuf.at[b], sem_g[b]).wait()

    for b in range(2):
        _issue(b, b)

    def _pair(i, _):
        k0 = i * 2
        for b in range(2):
            k = k0 + b
            _wait(k, b)

            @pl.when(k >= 2)
            def _(k=k, b=b):
                pltpu.make_async_copy(
                    mbuf.at[b], acc_sh.at[dst_all.at[k, 0]], sem_s[b]).wait()

            @plsc.parallel_loop(0, C, unroll=5)
            def _edge(c, b=b):
                for j in range(2):
                    fo = pl.ds(j * 16, 16)
                    so = pl.ds(32 + j * 16, 16)
                    lf = pbuf[b, c, fo] + qbuf[b, c, fo] + rbuf[b, c, fo]
                    ls = pbuf[b, c, so] + qbuf[b, c, so] + rbuf[b, c, so]
                    f = 1.0 / (1.0 + jnp.exp(-lf))
                    y = jnp.exp(-jnp.abs(ls))
                    t = y / (2.0 + y)
                    t2 = t * t
                    sp = jnp.maximum(ls, 0.0) + 2.0 * t * (
                        1.0 + t2 * (1.0 / 3.0 + t2 * (0.2 + t2 * (1.0 / 7.0))))
                    mbuf[b, c, fo] = f * sp

            pltpu.async_copy(
                mbuf.at[b], acc_sh.at[dst_all.at[k, 0]], sem_s[b], add=True)

            @pl.when(k + 2 < K)
            def _(k=k, b=b):
                _issue(k + 2, b)
        return 0

    lax.fori_loop(0, K // 2, _pair, 0)
    for b in range(2):
        pltpu.make_async_copy(
            mbuf.at[b], acc_sh.at[dst_all.at[K - 2 + b, 0]], sem_s[b]).wait()
    plsc.subcore_barrier()
    sl = pl.ds(sid * ROWS_PER_TILE, ROWS_PER_TILE)
    pltpu.sync_copy(acc_sh.at[sl], out_hbm.at[cid, sid])


_msgpass = functools.partial(
    pl.kernel,
    _msgpass_body,
    out_type=jax.ShapeDtypeStruct((NC, NS, ROWS_PER_TILE, D_H), jnp.float32),
    mesh=plsc.VectorSubcoreMesh(core_axis_name="c", subcore_axis_name="s"),
    scratch_types=[
        pltpu.VMEM((K, 1, C), jnp.int32),
        pltpu.VMEM((K, 1, C), jnp.int32),
        pltpu.VMEM((2, C, 2 * D_H), jnp.float32),
        pltpu.VMEM((2, C, 2 * D_H), jnp.float32),
        pltpu.VMEM((2, C, 2 * D_H), jnp.float32),
        pltpu.VMEM((2, C, D_H), jnp.float32),
        pltpu.VMEM((ROWS_PER_TILE, D_H), jnp.float32),
        pltpu.VMEM_SHARED((NPAD, D_H), jnp.float32),
        pltpu.SemaphoreType.DMA,
        pltpu.SemaphoreType.DMA,
        pltpu.SemaphoreType.DMA,
        pltpu.SemaphoreType.DMA,
    ],
    compiler_params=pltpu.CompilerParams(use_tc_tiling_on_sc=False,
                                         needs_layout_passes=True),
)()


@jax.jit
def kernel(x, edge_index, edge_attr, W_in, b_in, g_in, beta_in, W_e, b_e,
           g_e, beta_e, Wf0, bf0, Ws0, bs0, Wf1, bf1, Ws1, bs1, W_out,
           b_out, g_out, beta_out):
    f32 = jnp.float32
    b2 = lambda a: a.reshape(1, -1)

    h0 = pl.pallas_call(
        _h_embed_body,
        out_shape=jax.ShapeDtypeStruct((N, D_H), f32),
    )(x, W_in, b2(b_in), b2(g_in), b2(beta_in))

    dst3 = edge_index[1].reshape(NW * K, 1, C)
    src3 = edge_index[0].reshape(NW * K, 1, C)

    P8 = 8
    EC8 = 800
    a8 = edge_attr.reshape(E // P8, P8 * D_EDGE)
    G8, cs8 = pl.pallas_call(
        _egram_body,
        out_shape=[jax.ShapeDtypeStruct((P8 * D_EDGE, P8 * D_EDGE), f32),
                   jax.ShapeDtypeStruct((1, P8 * D_EDGE), f32)],
        grid=(E // P8 // EC8,),
        in_specs=[pl.BlockSpec((EC8, P8 * D_EDGE), lambda i: (i, 0))],
        out_specs=[pl.BlockSpec((P8 * D_EDGE, P8 * D_EDGE), lambda i: (0, 0)),
                   pl.BlockSpec((1, P8 * D_EDGE), lambda i: (0, 0))],
    )(a8)
    G = jnp.einsum('aiaj->ij', G8.reshape(P8, D_EDGE, P8, D_EDGE))
    cs = cs8.reshape(P8, D_EDGE).sum(axis=0)
    me = (cs @ W_e) / E + b_e
    Ey2 = (jnp.einsum('ij,ik,kj->j', W_e, G, W_e)
           + 2.0 * b_e * (cs @ W_e) + E * b_e * b_e) / E
    ve = Ey2 - me * me
    scale = g_e / jnp.sqrt(ve + 1e-5)
    We_t = W_e * scale
    be_t = (b_e - me) * scale + beta_e

    eye8 = jnp.eye(P8, dtype=f32)
    W8 = jnp.kron(eye8, We_t)
    b8 = jnp.tile(be_t, P8)
    w0 = jnp.kron(eye8, jnp.concatenate([Wf0[64:96], Ws0[64:96]], axis=1)).astype(jnp.bfloat16)
    b0 = jnp.tile(jnp.concatenate([bf0, bs0]), P8)
    w1 = jnp.kron(eye8, jnp.concatenate([Wf1[64:96], Ws1[64:96]], axis=1)).astype(jnp.bfloat16)
    b1 = jnp.tile(jnp.concatenate([bf1, bs1]), P8)

    def _table(w, b):
        r = pl.pallas_call(
            _edge_table_body,
            out_shape=jax.ShapeDtypeStruct((E // P8, P8 * 2 * D_H), f32),
            grid=(E // P8 // EC8,),
            in_specs=[
                pl.BlockSpec((EC8, P8 * D_EDGE), lambda i: (i, 0)),
                pl.BlockSpec((P8 * D_EDGE, P8 * D_H), lambda i: (0, 0)),
                pl.BlockSpec((1, P8 * D_H), lambda i: (0, 0)),
                pl.BlockSpec((P8 * D_H, P8 * 2 * D_H), lambda i: (0, 0)),
                pl.BlockSpec((1, P8 * 2 * D_H), lambda i: (0, 0)),
            ],
            out_specs=pl.BlockSpec((EC8, P8 * 2 * D_H), lambda i: (i, 0)),
        )(a8, W8, b2(b8), w, b2(b))
        return r.reshape(NW * K, C, 2 * D_H)

    R0 = _table(w0, b0)
    R1 = _table(w1, b1)

    wp0 = jnp.concatenate([Wf0[0:32], Ws0[0:32]], axis=1)
    wq0 = jnp.concatenate([Wf0[32:64], Ws0[32:64]], axis=1)
    P0, Q0 = pl.pallas_call(
        _pq0_body,
        out_shape=[jax.ShapeDtypeStruct((N, 2 * D_H), f32)] * 2,
    )(h0, wp0, wq0)
    acc0 = _msgpass(P0, Q0, R0, dst3, src3).reshape(NC, NPAD, D_H)

    wp1 = jnp.concatenate([Wf1[0:32], Ws1[0:32]], axis=1)
    wq1 = jnp.concatenate([Wf1[32:64], Ws1[32:64]], axis=1)
    h1, P1, Q1 = pl.pallas_call(
        _pq1_body,
        out_shape=[jax.ShapeDtypeStruct((N, D_H), f32)] +
                  [jax.ShapeDtypeStruct((N, 2 * D_H), f32)] * 2,
    )(h0, acc0, wp1, wq1)
    acc1 = _msgpass(P1, Q1, R1, dst3, src3).reshape(NC, NPAD, D_H)

    out = pl.pallas_call(
        _out_body,
        out_shape=jax.ShapeDtypeStruct((N, D_OUT), f32),
    )(h1, acc1, W_out, b2(b_out), b2(g_out), b2(beta_out))
    return out
